# baseline (speedup 1.0000x reference)
import jax
import jax.numpy as jnp
from jax.experimental import pallas as pl
from jax.experimental.pallas import tpu as pltpu

_MB = 4096


def _bin_embed_kernel(x_ref, low_ref, high_ref, w_ref, out_ref):
    K, D = w_ref.shape
    ii = jax.lax.broadcasted_iota(jnp.int32, (K, K), 0)
    kk = jax.lax.broadcasted_iota(jnp.int32, (K, K), 1)
    s = 1.0 / (jnp.abs(ii - kk) + 1).astype(jnp.float32)
    t = jnp.dot(s, w_ref[...], preferred_element_type=jnp.float32)

    x = x_ref[...]
    low = low_ref[...]
    high = high_ref[...]
    m = (x > low) & (x <= high)
    oh = jnp.where(m, 1.0, 0.0)
    out_ref[...] = jnp.dot(oh, t, preferred_element_type=jnp.float32)


def kernel(x, low, high, weight):
    B, F = x.shape
    K, D = weight.shape
    R = B * F
    xf = x.reshape(R, 1)
    low2 = low.reshape(1, K)
    high2 = high.reshape(1, K)

    out = pl.pallas_call(
        _bin_embed_kernel,
        grid=(R // _MB,),
        in_specs=[
            pl.BlockSpec((_MB, 1), lambda i: (i, 0)),
            pl.BlockSpec((1, K), lambda i: (0, 0)),
            pl.BlockSpec((1, K), lambda i: (0, 0)),
            pl.BlockSpec((K, D), lambda i: (0, 0)),
        ],
        out_specs=pl.BlockSpec((_MB, D), lambda i: (i, 0)),
        out_shape=jax.ShapeDtypeStruct((R, D), jnp.float32),
        compiler_params=pltpu.CompilerParams(
            dimension_semantics=("parallel",),
        ),
    )(xf, low2, high2, weight)
    return out.reshape(B, F, D)


# speedup vs baseline: 2.2205x; 2.2205x over previous
import jax
import jax.numpy as jnp
from jax.experimental import pallas as pl
from jax.experimental.pallas import tpu as pltpu

_BB = 64


def _bin_embed_kernel(x_ref, low_ref, high_ref, w_ref, out_ref):
    K, D = w_ref.shape
    BB, _, F = x_ref.shape
    ii = jax.lax.broadcasted_iota(jnp.int32, (K, K), 0)
    kk = jax.lax.broadcasted_iota(jnp.int32, (K, K), 1)
    s = 1.0 / (jnp.abs(ii - kk) + 1).astype(jnp.float32)
    t = jnp.dot(s, w_ref[...], preferred_element_type=jnp.float32)

    x = x_ref[...]
    low = low_ref[...]
    high = high_ref[...]
    m = (x > low) & (x <= high)
    oh = jnp.where(m, 1.0, 0.0)
    oht = jnp.swapaxes(oh, 1, 2)
    ohf = oht.reshape(BB * F, K)
    out_ref[...] = jnp.dot(ohf, t, preferred_element_type=jnp.float32)


def kernel(x, low, high, weight):
    B, F = x.shape
    K, D = weight.shape
    R = B * F
    x3 = x.reshape(B, 1, F)
    lowT = jnp.broadcast_to(low[:, None], (K, F)).reshape(1, K, F)
    highT = jnp.broadcast_to(high[:, None], (K, F)).reshape(1, K, F)

    out = pl.pallas_call(
        _bin_embed_kernel,
        grid=(B // _BB,),
        in_specs=[
            pl.BlockSpec((_BB, 1, F), lambda i: (i, 0, 0)),
            pl.BlockSpec((1, K, F), lambda i: (0, 0, 0)),
            pl.BlockSpec((1, K, F), lambda i: (0, 0, 0)),
            pl.BlockSpec((K, D), lambda i: (0, 0)),
        ],
        out_specs=pl.BlockSpec((_BB * F, D), lambda i: (i, 0)),
        out_shape=jax.ShapeDtypeStruct((R, D), jnp.float32),
        compiler_params=pltpu.CompilerParams(
            dimension_semantics=("parallel",),
        ),
    )(x3, lowT, highT, weight)
    return out.reshape(B, F, D)
